# Initial kernel scaffold; baseline (speedup 1.0000x reference)
import jax
import jax.numpy as jnp
from jax.experimental import pallas as pl


def kernel(h_V, h_E, center_id, batch_id, WV1, bV1, WV2, bV2, WV3, bV3, WB1, bB1, WB2, bB2, WB3, bB3, WO):
    raise NotImplementedError("write your pallas kernel here")


# speedup vs baseline: 31.5194x; 31.5194x over previous
import math

import jax
import jax.numpy as jnp
from jax import lax
from jax.experimental import pallas as pl
from jax.experimental.pallas import tpu as pltpu
from jax.experimental.pallas import tpu_sc as plsc

N_NODES = 10000
N_EDGES = 320000
H_DIM = 128
NIN_DIM = 256
N_HEADS = 4
D_HEAD = H_DIM // N_HEADS
SCALE = 1.0 / math.sqrt(D_HEAD)

NC = 2
NS = 16
NW = NC * NS
EPW = N_EDGES // NW
GC = 80
NCHUNK = EPW // GC

NPAD = 10240
ND = NPAD // 8
M_ROWS = N_NODES + ND
RPS = 704
WCH = 64
TAIL = M_ROWS - RPS * NS


def _gelu(x):
    return 0.5 * x * (1.0 + lax.erf(x * 0.7071067811865476))


def _p_body(hv_ref, w_ref, p_ref):
    p_ref[...] = jnp.dot(hv_ref[...], w_ref[...],
                         preferred_element_type=jnp.float32)


def _compute_p(h_V, WB1v):
    return pl.pallas_call(
        _p_body,
        out_shape=jax.ShapeDtypeStruct((N_NODES, H_DIM), jnp.float32),
    )(h_V, WB1v)


def _gather_body(p_hbm, idx_hbm, g_hbm, idx_v, rows_v, sem):
    c = lax.axis_index("c")
    s = lax.axis_index("s")
    wid = c * NS + s

    def chunk(i, _):
        off = wid * EPW + i * GC
        pltpu.sync_copy(idx_hbm.at[pl.ds(off, GC)], idx_v.at[0])
        pltpu.async_copy(p_hbm.at[idx_v.at[0]], rows_v, sem).wait()
        pltpu.sync_copy(rows_v, g_hbm.at[pl.ds(off, GC)])
        return 0

    lax.fori_loop(0, NCHUNK, chunk, 0)


def _gather(P, idx):
    mesh = plsc.VectorSubcoreMesh(core_axis_name="c", subcore_axis_name="s")
    f = pl.kernel(
        _gather_body,
        out_type=jax.ShapeDtypeStruct((N_EDGES, H_DIM), jnp.float32),
        mesh=mesh,
        scratch_types=[
            pltpu.VMEM((1, GC), jnp.int32),
            pltpu.VMEM((GC, H_DIM), jnp.float32),
            pltpu.SemaphoreType.DMA,
        ],
    )
    return f(P, idx)


def _edge_body(he_ref, g_ref, ids_ref, wb1e, bb1, wb2, bb2, wb3, bb3,
               wv1, bv1, wv2, bv2, wv3, bv3, expm, expm2, r_ref, d2_ref):
    x = he_ref[...]
    u = _gelu(jnp.dot(x, wb1e[...], preferred_element_type=jnp.float32)
              + g_ref[...] + bb1[...])
    u = _gelu(jnp.dot(u, wb2[...], preferred_element_type=jnp.float32)
              + bb2[...])
    w = (jnp.dot(u, wb3[...], preferred_element_type=jnp.float32)
         + bb3[...]) * SCALE
    lanes = lax.broadcasted_iota(jnp.int32, w.shape, 1)
    ew = jnp.where(lanes < N_HEADS, jnp.exp(w), 0.0)
    ids = ids_ref[0, 0, :].reshape(-1, 1)
    blk = lax.broadcasted_iota(jnp.int32, (ids.shape[0], H_DIM), 1) // 16
    mask = (blk == (ids % 8)).astype(jnp.float32)
    d2_ref[...] = jnp.dot(ew, expm2[...],
                          preferred_element_type=jnp.float32) * mask
    v = _gelu(jnp.dot(x, wv1[...], preferred_element_type=jnp.float32)
              + bv1[...])
    v = _gelu(jnp.dot(v, wv2[...], preferred_element_type=jnp.float32)
              + bv2[...])
    v = jnp.dot(v, wv3[...], preferred_element_type=jnp.float32) + bv3[...]
    r_ref[...] = v * jnp.dot(ew, expm[...], preferred_element_type=jnp.float32)


def _edge_stage(h_E, G, ids3, WB1e, bB1, WB2, bB2, WB3p, bB3p,
                WV1, bV1, WV2, bV2, WV3, bV3, EXPM, EXPM2):
    BE = 2560
    grid = (N_EDGES // BE,)
    wspec = lambda shape: pl.BlockSpec(shape, lambda i: tuple(0 for _ in shape))
    return pl.pallas_call(
        _edge_body,
        grid=grid,
        in_specs=[
            pl.BlockSpec((BE, NIN_DIM), lambda i: (i, 0)),
            pl.BlockSpec((BE, H_DIM), lambda i: (i, 0)),
            pl.BlockSpec((1, 1, BE), lambda i: (i, 0, 0)),
            wspec((NIN_DIM, H_DIM)), wspec((1, H_DIM)),
            wspec((H_DIM, H_DIM)), wspec((1, H_DIM)),
            wspec((H_DIM, 16)), wspec((1, 16)),
            wspec((NIN_DIM, H_DIM)), wspec((1, H_DIM)),
            wspec((H_DIM, H_DIM)), wspec((1, H_DIM)),
            wspec((H_DIM, H_DIM)), wspec((1, H_DIM)),
            wspec((16, H_DIM)), wspec((16, H_DIM)),
        ],
        out_specs=[
            pl.BlockSpec((BE, H_DIM), lambda i: (i, 0)),
            pl.BlockSpec((BE, H_DIM), lambda i: (i, 0)),
        ],
        out_shape=[
            jax.ShapeDtypeStruct((N_EDGES, H_DIM), jnp.float32),
            jax.ShapeDtypeStruct((N_EDGES, H_DIM), jnp.float32),
        ],
    )(h_E, G, ids3, WB1e, bB1, WB2, bB2, WB3p, bB3p,
      WV1, bV1, WV2, bV2, WV3, bV3, EXPM, EXPM2)


def _scatter_body(r_hbm, d2_hbm, idx_hbm, pn_hbm,
                  idxn_v, idxd_v, r_v, d_v, acc):
    c = lax.axis_index("c")
    s = lax.axis_index("s")
    wid = c * NS + s

    def zfill(r, _):
        for j in range(8):
            r_v[r, 16 * j:16 * (j + 1)] = jnp.zeros((16,), jnp.float32)
        return 0

    lax.fori_loop(0, WCH, zfill, 0)

    def zinit(j, _):
        pltpu.sync_copy(r_v.at[pl.ds(0, WCH)],
                        acc.at[pl.ds(s * RPS + j * WCH, WCH)])
        return 0

    lax.fori_loop(0, RPS // WCH, zinit, 0)

    @pl.when(s == 0)
    def _():
        pltpu.sync_copy(r_v.at[pl.ds(0, TAIL)], acc.at[pl.ds(NS * RPS, TAIL)])

    plsc.subcore_barrier()

    def chunk(i, _):
        off = wid * EPW + i * GC
        pltpu.sync_copy(idx_hbm.at[pl.ds(off, GC)], idxn_v.at[0])
        for k in range(GC // 16):
            v = idxn_v[0, 16 * k:16 * (k + 1)]
            idxd_v[0, 16 * k:16 * (k + 1)] = N_NODES + (v >> 3)
        pltpu.sync_copy(r_hbm.at[pl.ds(off, GC)], r_v)
        pltpu.sync_copy(d2_hbm.at[pl.ds(off, GC)], d_v)
        pltpu.sync_copy(r_v, acc.at[idxn_v.at[0]], add=True)
        pltpu.sync_copy(d_v, acc.at[idxd_v.at[0]], add=True)
        return 0

    lax.fori_loop(0, NCHUNK, chunk, 0)
    plsc.subcore_barrier()

    def wout(j, _):
        row = s * RPS + j * WCH
        pltpu.sync_copy(acc.at[pl.ds(row, WCH)], r_v.at[pl.ds(0, WCH)])
        pltpu.sync_copy(r_v.at[pl.ds(0, WCH)],
                        pn_hbm.at[pl.ds(c * M_ROWS + row, WCH)])
        return 0

    lax.fori_loop(0, RPS // WCH, wout, 0)

    @pl.when(s == 0)
    def _():
        pltpu.sync_copy(acc.at[pl.ds(NS * RPS, TAIL)], r_v.at[pl.ds(0, TAIL)])
        pltpu.sync_copy(r_v.at[pl.ds(0, TAIL)],
                        pn_hbm.at[pl.ds(c * M_ROWS + NS * RPS, TAIL)])


def _scatter(R, D2, idx):
    mesh = plsc.VectorSubcoreMesh(core_axis_name="c", subcore_axis_name="s")
    f = pl.kernel(
        _scatter_body,
        out_type=jax.ShapeDtypeStruct((NC * M_ROWS, H_DIM), jnp.float32),
        mesh=mesh,
        scratch_types=[
            pltpu.VMEM((1, GC), jnp.int32),
            pltpu.VMEM((1, GC), jnp.int32),
            pltpu.VMEM((GC, H_DIM), jnp.float32),
            pltpu.VMEM((GC, H_DIM), jnp.float32),
            pltpu.VMEM_SHARED((M_ROWS, H_DIM), jnp.float32),
        ],
    )
    return f(R, D2, idx)


def _combine_body(pn_ref, d16_ref, wo_ref, expm_ref, out_ref):
    n = pn_ref[0:N_NODES, :] + pn_ref[M_ROWS:M_ROWS + N_NODES, :]
    d = d16_ref[0, 0:N_NODES, :] + d16_ref[1, 0:N_NODES, :]
    r = jnp.where(d > 0.0, 1.0 / d, 0.0)
    h = n * jnp.dot(r, expm_ref[...], preferred_element_type=jnp.float32)
    out_ref[...] = jnp.dot(h, wo_ref[...], preferred_element_type=jnp.float32)


def _combine(pn, d16, WO, EXPM):
    return pl.pallas_call(
        _combine_body,
        out_shape=jax.ShapeDtypeStruct((N_NODES, H_DIM), jnp.float32),
    )(pn, d16, WO, EXPM)


@jax.jit
def kernel(h_V, h_E, center_id, batch_id,
           WV1, bV1, WV2, bV2, WV3, bV3,
           WB1, bB1, WB2, bB2, WB3, bB3, WO):
    WB1v = WB1[:H_DIM]
    WB1e = WB1[H_DIM:]
    WB3p = jnp.pad(WB3, ((0, 0), (0, 16 - N_HEADS)))
    bB3p = jnp.pad(bB3, (0, 16 - N_HEADS)).reshape(1, 16)
    head16 = jnp.arange(16, dtype=jnp.int32)[:, None]
    col = jnp.arange(H_DIM, dtype=jnp.int32)[None, :]
    EXPM = (head16 == col // D_HEAD).astype(jnp.float32)
    EXPM2 = (head16 == col % 16).astype(jnp.float32)
    ids3 = center_id.reshape(N_EDGES // 2560, 1, 2560)

    P = _compute_p(h_V, WB1v)
    G = _gather(P, center_id)
    R, D2 = _edge_stage(h_E, G, ids3, WB1e, bB1.reshape(1, H_DIM),
                        WB2, bB2.reshape(1, H_DIM), WB3p, bB3p,
                        WV1, bV1.reshape(1, H_DIM), WV2, bV2.reshape(1, H_DIM),
                        WV3, bV3.reshape(1, H_DIM), EXPM, EXPM2)
    pn = _scatter(R, D2, center_id)
    d16 = jnp.stack([pn[N_NODES:N_NODES + ND],
                     pn[M_ROWS + N_NODES:M_ROWS + N_NODES + ND]]
                    ).reshape(NC, NPAD, 16)
    return _combine(pn, d16, WO, EXPM)


# speedup vs baseline: 43.9297x; 1.3937x over previous
import math

import jax
import jax.numpy as jnp
from jax import lax
from jax.experimental import pallas as pl
from jax.experimental.pallas import tpu as pltpu
from jax.experimental.pallas import tpu_sc as plsc

N_NODES = 10000
N_EDGES = 320000
H_DIM = 128
NIN_DIM = 256
N_HEADS = 4
D_HEAD = H_DIM // N_HEADS
SCALE = 1.0 / math.sqrt(D_HEAD)

NC = 2
NS = 16
NW = NC * NS
EPW = N_EDGES // NW
GC = 80
NCHUNK = EPW // GC

NPAD = 10240
ND = NPAD // 8
M_ROWS = N_NODES + ND
RPS = 704
WCH = 64
TAIL = M_ROWS - RPS * NS


def _gelu(x):
    return 0.5 * x * (1.0 + lax.erf(x * 0.7071067811865476))


def _p_body(hv_ref, w_ref, p_ref):
    p_ref[...] = jnp.dot(hv_ref[...], w_ref[...],
                         preferred_element_type=jnp.float32)


def _compute_p(h_V, WB1v):
    return pl.pallas_call(
        _p_body,
        out_shape=jax.ShapeDtypeStruct((N_NODES, H_DIM), jnp.float32),
    )(h_V, WB1v)


KG = 8
BLK = KG * GC
NBLK = N_EDGES // BLK
GITER = (NBLK + NW - 1) // NW


def _gather_body(p_hbm, idx2_hbm, g_hbm, idx_v, rows_v, sem):
    c = lax.axis_index("c")
    s = lax.axis_index("s")
    wid = c * NS + s

    def chunk(t, _):
        b = wid + NW * t

        @pl.when(b < NBLK)
        def _():
            pltpu.sync_copy(idx2_hbm.at[pl.ds(b * KG, KG)], idx_v)
            hs = [pltpu.async_copy(p_hbm.at[idx_v.at[j]],
                                   rows_v.at[pl.ds(j * GC, GC)], sem)
                  for j in range(KG)]
            for h in hs:
                h.wait()
            pltpu.sync_copy(rows_v, g_hbm.at[pl.ds(b * BLK, BLK)])

        return 0

    lax.fori_loop(0, GITER, chunk, 0)


def _gather(P, idx2):
    mesh = plsc.VectorSubcoreMesh(core_axis_name="c", subcore_axis_name="s")
    f = pl.kernel(
        _gather_body,
        out_type=jax.ShapeDtypeStruct((N_EDGES, H_DIM), jnp.float32),
        mesh=mesh,
        scratch_types=[
            pltpu.VMEM((KG, GC), jnp.int32),
            pltpu.VMEM((BLK, H_DIM), jnp.float32),
            pltpu.SemaphoreType.DMA,
        ],
    )
    return f(P, idx2)


def _edge_body(he_ref, g_ref, ids_ref, wb1e, bb1, wb2, bb2, wb3, bb3,
               wv1, bv1, wv2, bv2, wv3, bv3, expm, expm2, r_ref, d2_ref):
    x = he_ref[...]
    u = _gelu(jnp.dot(x, wb1e[...], preferred_element_type=jnp.float32)
              + g_ref[...] + bb1[...])
    u = _gelu(jnp.dot(u, wb2[...], preferred_element_type=jnp.float32)
              + bb2[...])
    w = (jnp.dot(u, wb3[...], preferred_element_type=jnp.float32)
         + bb3[...]) * SCALE
    lanes = lax.broadcasted_iota(jnp.int32, w.shape, 1)
    ew = jnp.where(lanes < N_HEADS, jnp.exp(w), 0.0)
    ids = ids_ref[0, 0, :].reshape(-1, 1)
    blk = lax.broadcasted_iota(jnp.int32, (ids.shape[0], H_DIM), 1) // 16
    mask = (blk == (ids % 8)).astype(jnp.float32)
    d2_ref[...] = jnp.dot(ew, expm2[...],
                          preferred_element_type=jnp.float32) * mask
    v = _gelu(jnp.dot(x, wv1[...], preferred_element_type=jnp.float32)
              + bv1[...])
    v = _gelu(jnp.dot(v, wv2[...], preferred_element_type=jnp.float32)
              + bv2[...])
    v = jnp.dot(v, wv3[...], preferred_element_type=jnp.float32) + bv3[...]
    r_ref[...] = v * jnp.dot(ew, expm[...], preferred_element_type=jnp.float32)


def _edge_stage(h_E, G, ids3, WB1e, bB1, WB2, bB2, WB3p, bB3p,
                WV1, bV1, WV2, bV2, WV3, bV3, EXPM, EXPM2):
    BE = 2560
    grid = (N_EDGES // BE,)
    wspec = lambda shape: pl.BlockSpec(shape, lambda i: tuple(0 for _ in shape))
    return pl.pallas_call(
        _edge_body,
        grid=grid,
        in_specs=[
            pl.BlockSpec((BE, NIN_DIM), lambda i: (i, 0)),
            pl.BlockSpec((BE, H_DIM), lambda i: (i, 0)),
            pl.BlockSpec((1, 1, BE), lambda i: (i, 0, 0)),
            wspec((NIN_DIM, H_DIM)), wspec((1, H_DIM)),
            wspec((H_DIM, H_DIM)), wspec((1, H_DIM)),
            wspec((H_DIM, 16)), wspec((1, 16)),
            wspec((NIN_DIM, H_DIM)), wspec((1, H_DIM)),
            wspec((H_DIM, H_DIM)), wspec((1, H_DIM)),
            wspec((H_DIM, H_DIM)), wspec((1, H_DIM)),
            wspec((16, H_DIM)), wspec((16, H_DIM)),
        ],
        out_specs=[
            pl.BlockSpec((BE, H_DIM), lambda i: (i, 0)),
            pl.BlockSpec((BE, H_DIM), lambda i: (i, 0)),
        ],
        out_shape=[
            jax.ShapeDtypeStruct((N_EDGES, H_DIM), jnp.float32),
            jax.ShapeDtypeStruct((N_EDGES, H_DIM), jnp.float32),
        ],
    )(h_E, G, ids3, WB1e, bB1, WB2, bB2, WB3p, bB3p,
      WV1, bV1, WV2, bV2, WV3, bV3, EXPM, EXPM2)


def _scatter_body(r_hbm, d2_hbm, idx_hbm, pn_hbm,
                  idxn_v, idxd_v, r_v, d_v, acc, sem1, sem2):
    c = lax.axis_index("c")
    s = lax.axis_index("s")
    wid = c * NS + s

    def zfill(r, _):
        for j in range(8):
            r_v[r, 16 * j:16 * (j + 1)] = jnp.zeros((16,), jnp.float32)
        return 0

    lax.fori_loop(0, WCH, zfill, 0)

    def zinit(j, _):
        pltpu.sync_copy(r_v.at[pl.ds(0, WCH)],
                        acc.at[pl.ds(s * RPS + j * WCH, WCH)])
        return 0

    lax.fori_loop(0, RPS // WCH, zinit, 0)

    @pl.when(s == 0)
    def _():
        pltpu.sync_copy(r_v.at[pl.ds(0, TAIL)], acc.at[pl.ds(NS * RPS, TAIL)])

    plsc.subcore_barrier()

    def chunk(i, _):
        off = wid * EPW + i * GC

        @pl.when(i > 0)
        def _():
            pltpu.make_async_copy(r_hbm.at[pl.ds(0, GC)], r_v, sem2).wait()
            pltpu.make_async_copy(d2_hbm.at[pl.ds(0, GC)], d_v, sem2).wait()

        h1 = pltpu.async_copy(idx_hbm.at[pl.ds(off, GC)], idxn_v.at[0], sem1)
        h2 = pltpu.async_copy(r_hbm.at[pl.ds(off, GC)], r_v, sem1)
        h3 = pltpu.async_copy(d2_hbm.at[pl.ds(off, GC)], d_v, sem1)
        h1.wait()
        h2.wait()
        h3.wait()
        for k in range(GC // 16):
            v = idxn_v[0, 16 * k:16 * (k + 1)]
            idxd_v[0, 16 * k:16 * (k + 1)] = N_NODES + (v >> 3)
        pltpu.async_copy(r_v, acc.at[idxn_v.at[0]], sem2, add=True)
        pltpu.async_copy(d_v, acc.at[idxd_v.at[0]], sem2, add=True)
        return 0

    lax.fori_loop(0, NCHUNK, chunk, 0)
    pltpu.make_async_copy(r_hbm.at[pl.ds(0, GC)], r_v, sem2).wait()
    pltpu.make_async_copy(d2_hbm.at[pl.ds(0, GC)], d_v, sem2).wait()
    plsc.subcore_barrier()

    def wout(j, _):
        row = s * RPS + j * WCH
        pltpu.sync_copy(acc.at[pl.ds(row, WCH)], r_v.at[pl.ds(0, WCH)])
        pltpu.sync_copy(r_v.at[pl.ds(0, WCH)],
                        pn_hbm.at[pl.ds(c * M_ROWS + row, WCH)])
        return 0

    lax.fori_loop(0, RPS // WCH, wout, 0)

    @pl.when(s == 0)
    def _():
        pltpu.sync_copy(acc.at[pl.ds(NS * RPS, TAIL)], r_v.at[pl.ds(0, TAIL)])
        pltpu.sync_copy(r_v.at[pl.ds(0, TAIL)],
                        pn_hbm.at[pl.ds(c * M_ROWS + NS * RPS, TAIL)])


def _scatter(R, D2, idx):
    mesh = plsc.VectorSubcoreMesh(core_axis_name="c", subcore_axis_name="s")
    f = pl.kernel(
        _scatter_body,
        out_type=jax.ShapeDtypeStruct((NC * M_ROWS, H_DIM), jnp.float32),
        mesh=mesh,
        scratch_types=[
            pltpu.VMEM((1, GC), jnp.int32),
            pltpu.VMEM((1, GC), jnp.int32),
            pltpu.VMEM((GC, H_DIM), jnp.float32),
            pltpu.VMEM((GC, H_DIM), jnp.float32),
            pltpu.VMEM_SHARED((M_ROWS, H_DIM), jnp.float32),
            pltpu.SemaphoreType.DMA,
            pltpu.SemaphoreType.DMA,
        ],
    )
    return f(R, D2, idx)


def _combine_body(pn_ref, d16_ref, wo_ref, expm_ref, out_ref):
    n = pn_ref[0:N_NODES, :] + pn_ref[M_ROWS:M_ROWS + N_NODES, :]
    d = d16_ref[0, 0:N_NODES, :] + d16_ref[1, 0:N_NODES, :]
    r = jnp.where(d > 0.0, 1.0 / d, 0.0)
    h = n * jnp.dot(r, expm_ref[...], preferred_element_type=jnp.float32)
    out_ref[...] = jnp.dot(h, wo_ref[...], preferred_element_type=jnp.float32)


def _combine(pn, d16, WO, EXPM):
    return pl.pallas_call(
        _combine_body,
        out_shape=jax.ShapeDtypeStruct((N_NODES, H_DIM), jnp.float32),
    )(pn, d16, WO, EXPM)


@jax.jit
def kernel(h_V, h_E, center_id, batch_id,
           WV1, bV1, WV2, bV2, WV3, bV3,
           WB1, bB1, WB2, bB2, WB3, bB3, WO):
    WB1v = WB1[:H_DIM]
    WB1e = WB1[H_DIM:]
    WB3p = jnp.pad(WB3, ((0, 0), (0, 16 - N_HEADS)))
    bB3p = jnp.pad(bB3, (0, 16 - N_HEADS)).reshape(1, 16)
    head16 = jnp.arange(16, dtype=jnp.int32)[:, None]
    col = jnp.arange(H_DIM, dtype=jnp.int32)[None, :]
    EXPM = (head16 == col // D_HEAD).astype(jnp.float32)
    EXPM2 = (head16 == col % 16).astype(jnp.float32)
    ids3 = center_id.reshape(N_EDGES // 2560, 1, 2560)
    idx2 = center_id.reshape(N_EDGES // GC, GC)

    P = _compute_p(h_V, WB1v)
    G = _gather(P, idx2)
    R, D2 = _edge_stage(h_E, G, ids3, WB1e, bB1.reshape(1, H_DIM),
                        WB2, bB2.reshape(1, H_DIM), WB3p, bB3p,
                        WV1, bV1.reshape(1, H_DIM), WV2, bV2.reshape(1, H_DIM),
                        WV3, bV3.reshape(1, H_DIM), EXPM, EXPM2)
    pn = _scatter(R, D2, center_id)
    d16 = jnp.stack([pn[N_NODES:N_NODES + ND],
                     pn[M_ROWS + N_NODES:M_ROWS + N_NODES + ND]]
                    ).reshape(NC, NPAD, 16)
    return _combine(pn, d16, WO, EXPM)


# speedup vs baseline: 44.4449x; 1.0117x over previous
import math

import jax
import jax.numpy as jnp
from jax import lax
from jax.experimental import pallas as pl
from jax.experimental.pallas import tpu as pltpu
from jax.experimental.pallas import tpu_sc as plsc

N_NODES = 10000
N_EDGES = 320000
H_DIM = 128
NIN_DIM = 256
N_HEADS = 4
D_HEAD = H_DIM // N_HEADS
SCALE = 1.0 / math.sqrt(D_HEAD)

NC = 2
NS = 16
NW = NC * NS
EPW = N_EDGES // NW
GC = 80
NCHUNK = EPW // GC

NPAD = 10240
ND = NPAD // 8
M_ROWS = N_NODES + ND
RPS = 704
WCH = 32
TAIL = M_ROWS - RPS * NS


def _gelu(x):
    return 0.5 * x * (1.0 + lax.erf(x * 0.7071067811865476))


def _p_body(hv_ref, w_ref, p_ref):
    p_ref[...] = jnp.dot(hv_ref[...], w_ref[...],
                         preferred_element_type=jnp.float32)


def _compute_p(h_V, WB1v):
    return pl.pallas_call(
        _p_body,
        out_shape=jax.ShapeDtypeStruct((N_NODES, H_DIM), jnp.float32),
    )(h_V, WB1v)


KG = 8
BLK = KG * GC
NBLK = N_EDGES // BLK
GITER = (NBLK + NW - 1) // NW


def _gather_body(p_hbm, idx2_hbm, g_hbm, idx_v, rows_v, sem):
    c = lax.axis_index("c")
    s = lax.axis_index("s")
    wid = c * NS + s

    def chunk(t, _):
        b = wid + NW * t

        @pl.when(b < NBLK)
        def _():
            pltpu.sync_copy(idx2_hbm.at[pl.ds(b * KG, KG)], idx_v)
            hs = [pltpu.async_copy(p_hbm.at[idx_v.at[j]],
                                   rows_v.at[pl.ds(j * GC, GC)], sem)
                  for j in range(KG)]
            for h in hs:
                h.wait()
            pltpu.sync_copy(rows_v, g_hbm.at[pl.ds(b * BLK, BLK)])

        return 0

    lax.fori_loop(0, GITER, chunk, 0)


def _gather(P, idx2):
    mesh = plsc.VectorSubcoreMesh(core_axis_name="c", subcore_axis_name="s")
    f = pl.kernel(
        _gather_body,
        out_type=jax.ShapeDtypeStruct((N_EDGES, H_DIM), jnp.float32),
        mesh=mesh,
        scratch_types=[
            pltpu.VMEM((KG, GC), jnp.int32),
            pltpu.VMEM((BLK, H_DIM), jnp.float32),
            pltpu.SemaphoreType.DMA,
        ],
    )
    return f(P, idx2)


def _edge_body(he_ref, g_ref, ids_ref, wb1e, bb1, wb2, bb2, wb3, bb3,
               wv1, bv1, wv2, bv2, wv3, bv3, expm, expm2, r_ref, d2_ref):
    x = he_ref[...]
    u = _gelu(jnp.dot(x, wb1e[...], preferred_element_type=jnp.float32)
              + g_ref[...] + bb1[...])
    u = _gelu(jnp.dot(u, wb2[...], preferred_element_type=jnp.float32)
              + bb2[...])
    w = (jnp.dot(u, wb3[...], preferred_element_type=jnp.float32)
         + bb3[...]) * SCALE
    lanes = lax.broadcasted_iota(jnp.int32, w.shape, 1)
    ew = jnp.where(lanes < N_HEADS, jnp.exp(w), 0.0)
    ids = ids_ref[0, 0, :].reshape(-1, 1)
    blk = lax.broadcasted_iota(jnp.int32, (ids.shape[0], H_DIM), 1) // 16
    mask = (blk == (ids % 8)).astype(jnp.float32)
    d2_ref[...] = jnp.dot(ew, expm2[...],
                          preferred_element_type=jnp.float32) * mask
    v = _gelu(jnp.dot(x, wv1[...], preferred_element_type=jnp.float32)
              + bv1[...])
    v = _gelu(jnp.dot(v, wv2[...], preferred_element_type=jnp.float32)
              + bv2[...])
    v = jnp.dot(v, wv3[...], preferred_element_type=jnp.float32) + bv3[...]
    r_ref[...] = v * jnp.dot(ew, expm[...], preferred_element_type=jnp.float32)


def _edge_stage(h_E, G, ids3, WB1e, bB1, WB2, bB2, WB3p, bB3p,
                WV1, bV1, WV2, bV2, WV3, bV3, EXPM, EXPM2):
    BE = 2560
    grid = (N_EDGES // BE,)
    wspec = lambda shape: pl.BlockSpec(shape, lambda i: tuple(0 for _ in shape))
    return pl.pallas_call(
        _edge_body,
        grid=grid,
        in_specs=[
            pl.BlockSpec((BE, NIN_DIM), lambda i: (i, 0)),
            pl.BlockSpec((BE, H_DIM), lambda i: (i, 0)),
            pl.BlockSpec((1, 1, BE), lambda i: (i, 0, 0)),
            wspec((NIN_DIM, H_DIM)), wspec((1, H_DIM)),
            wspec((H_DIM, H_DIM)), wspec((1, H_DIM)),
            wspec((H_DIM, 16)), wspec((1, 16)),
            wspec((NIN_DIM, H_DIM)), wspec((1, H_DIM)),
            wspec((H_DIM, H_DIM)), wspec((1, H_DIM)),
            wspec((H_DIM, H_DIM)), wspec((1, H_DIM)),
            wspec((16, H_DIM)), wspec((16, H_DIM)),
        ],
        out_specs=[
            pl.BlockSpec((BE, H_DIM), lambda i: (i, 0)),
            pl.BlockSpec((BE, H_DIM), lambda i: (i, 0)),
        ],
        out_shape=[
            jax.ShapeDtypeStruct((N_EDGES, H_DIM), jnp.float32),
            jax.ShapeDtypeStruct((N_EDGES, H_DIM), jnp.float32),
        ],
    )(h_E, G, ids3, WB1e, bB1, WB2, bB2, WB3p, bB3p,
      WV1, bV1, WV2, bV2, WV3, bV3, EXPM, EXPM2)


GC2 = 40
NPAIR = EPW // (2 * GC2)


def _scatter_body(r_hbm, d2_hbm, idx_hbm, pn_hbm,
                  idxn_vA, idxd_vA, r_vA, d_vA,
                  idxn_vB, idxd_vB, r_vB, d_vB,
                  acc, semLA, semLB, semAA, semAB):
    c = lax.axis_index("c")
    s = lax.axis_index("s")
    wid = c * NS + s

    def zfill(r, _):
        for j in range(8):
            r_vA[r, 16 * j:16 * (j + 1)] = jnp.zeros((16,), jnp.float32)
        return 0

    lax.fori_loop(0, WCH, zfill, 0)

    def zinit(j, _):
        pltpu.sync_copy(r_vA.at[pl.ds(0, WCH)],
                        acc.at[pl.ds(s * RPS + j * WCH, WCH)])
        return 0

    lax.fori_loop(0, RPS // WCH, zinit, 0)

    @pl.when(s == 0)
    def _():
        pltpu.sync_copy(r_vA.at[pl.ds(0, TAIL)], acc.at[pl.ds(NS * RPS, TAIL)])

    plsc.subcore_barrier()

    def _idxd(idxn_v, idxd_v):
        for k0 in (0, 16, 24):
            v = idxn_v[0, k0:k0 + 16]
            idxd_v[0, k0:k0 + 16] = N_NODES + (v >> 3)

    def _drain(sem):
        pltpu.make_async_copy(r_hbm.at[pl.ds(0, GC2)], r_vA, sem).wait()
        pltpu.make_async_copy(d2_hbm.at[pl.ds(0, GC2)], d_vA, sem).wait()

    def chunk(t, _):
        offA = wid * EPW + (2 * t) * GC2
        offB = offA + GC2

        @pl.when(t > 0)
        def _():
            _drain(semAA)

        hA = [pltpu.async_copy(idx_hbm.at[pl.ds(offA, GC2)],
                               idxn_vA.at[0], semLA),
              pltpu.async_copy(r_hbm.at[pl.ds(offA, GC2)], r_vA, semLA),
              pltpu.async_copy(d2_hbm.at[pl.ds(offA, GC2)], d_vA, semLA)]

        @pl.when(t > 0)
        def _():
            _drain(semAB)

        hB = [pltpu.async_copy(idx_hbm.at[pl.ds(offB, GC2)],
                               idxn_vB.at[0], semLB),
              pltpu.async_copy(r_hbm.at[pl.ds(offB, GC2)], r_vB, semLB),
              pltpu.async_copy(d2_hbm.at[pl.ds(offB, GC2)], d_vB, semLB)]

        for h in hA:
            h.wait()
        _idxd(idxn_vA, idxd_vA)
        pltpu.async_copy(r_vA, acc.at[idxn_vA.at[0]], semAA, add=True)
        pltpu.async_copy(d_vA, acc.at[idxd_vA.at[0]], semAA, add=True)

        for h in hB:
            h.wait()
        _idxd(idxn_vB, idxd_vB)
        pltpu.async_copy(r_vB, acc.at[idxn_vB.at[0]], semAB, add=True)
        pltpu.async_copy(d_vB, acc.at[idxd_vB.at[0]], semAB, add=True)
        return 0

    lax.fori_loop(0, NPAIR, chunk, 0)
    _drain(semAA)
    _drain(semAB)
    plsc.subcore_barrier()

    def wout(j, _):
        row = s * RPS + j * WCH
        pltpu.sync_copy(acc.at[pl.ds(row, WCH)], r_vA.at[pl.ds(0, WCH)])
        pltpu.sync_copy(r_vA.at[pl.ds(0, WCH)],
                        pn_hbm.at[pl.ds(c * M_ROWS + row, WCH)])
        return 0

    lax.fori_loop(0, RPS // WCH, wout, 0)

    @pl.when(s == 0)
    def _():
        pltpu.sync_copy(acc.at[pl.ds(NS * RPS, TAIL)], r_vA.at[pl.ds(0, TAIL)])
        pltpu.sync_copy(r_vA.at[pl.ds(0, TAIL)],
                        pn_hbm.at[pl.ds(c * M_ROWS + NS * RPS, TAIL)])


def _scatter(R, D2, idx):
    mesh = plsc.VectorSubcoreMesh(core_axis_name="c", subcore_axis_name="s")
    f = pl.kernel(
        _scatter_body,
        out_type=jax.ShapeDtypeStruct((NC * M_ROWS, H_DIM), jnp.float32),
        mesh=mesh,
        scratch_types=[
            pltpu.VMEM((1, GC2), jnp.int32),
            pltpu.VMEM((1, GC2), jnp.int32),
            pltpu.VMEM((GC2, H_DIM), jnp.float32),
            pltpu.VMEM((GC2, H_DIM), jnp.float32),
            pltpu.VMEM((1, GC2), jnp.int32),
            pltpu.VMEM((1, GC2), jnp.int32),
            pltpu.VMEM((GC2, H_DIM), jnp.float32),
            pltpu.VMEM((GC2, H_DIM), jnp.float32),
            pltpu.VMEM_SHARED((M_ROWS, H_DIM), jnp.float32),
            pltpu.SemaphoreType.DMA,
            pltpu.SemaphoreType.DMA,
            pltpu.SemaphoreType.DMA,
            pltpu.SemaphoreType.DMA,
        ],
    )
    return f(R, D2, idx)


def _combine_body(pn_ref, d16_ref, wo_ref, expm_ref, out_ref):
    n = pn_ref[0:N_NODES, :] + pn_ref[M_ROWS:M_ROWS + N_NODES, :]
    d = d16_ref[0, 0:N_NODES, :] + d16_ref[1, 0:N_NODES, :]
    r = jnp.where(d > 0.0, 1.0 / d, 0.0)
    h = n * jnp.dot(r, expm_ref[...], preferred_element_type=jnp.float32)
    out_ref[...] = jnp.dot(h, wo_ref[...], preferred_element_type=jnp.float32)


def _combine(pn, d16, WO, EXPM):
    return pl.pallas_call(
        _combine_body,
        out_shape=jax.ShapeDtypeStruct((N_NODES, H_DIM), jnp.float32),
    )(pn, d16, WO, EXPM)


@jax.jit
def kernel(h_V, h_E, center_id, batch_id,
           WV1, bV1, WV2, bV2, WV3, bV3,
           WB1, bB1, WB2, bB2, WB3, bB3, WO):
    WB1v = WB1[:H_DIM]
    WB1e = WB1[H_DIM:]
    WB3p = jnp.pad(WB3, ((0, 0), (0, 16 - N_HEADS)))
    bB3p = jnp.pad(bB3, (0, 16 - N_HEADS)).reshape(1, 16)
    head16 = jnp.arange(16, dtype=jnp.int32)[:, None]
    col = jnp.arange(H_DIM, dtype=jnp.int32)[None, :]
    EXPM = (head16 == col // D_HEAD).astype(jnp.float32)
    EXPM2 = (head16 == col % 16).astype(jnp.float32)
    ids3 = center_id.reshape(N_EDGES // 2560, 1, 2560)
    idx2 = center_id.reshape(N_EDGES // GC, GC)

    P = _compute_p(h_V, WB1v)
    G = _gather(P, idx2)
    R, D2 = _edge_stage(h_E, G, ids3, WB1e, bB1.reshape(1, H_DIM),
                        WB2, bB2.reshape(1, H_DIM), WB3p, bB3p,
                        WV1, bV1.reshape(1, H_DIM), WV2, bV2.reshape(1, H_DIM),
                        WV3, bV3.reshape(1, H_DIM), EXPM, EXPM2)
    pn = _scatter(R, D2, center_id)
    d16 = jnp.stack([pn[N_NODES:N_NODES + ND],
                     pn[M_ROWS + N_NODES:M_ROWS + N_NODES + ND]]
                    ).reshape(NC, NPAD, 16)
    return _combine(pn, d16, WO, EXPM)
